# Initial kernel scaffold; baseline (speedup 1.0000x reference)
import jax
import jax.numpy as jnp
from jax.experimental import pallas as pl


def kernel(x, edge_index, edge_type, rel_weight, self_loop_weight, bias):
    raise NotImplementedError("write your pallas kernel here")


# speedup vs baseline: 37.8019x; 37.8019x over previous
import jax
import jax.numpy as jnp
from jax import lax
from jax.experimental import pallas as pl
from jax.experimental.pallas import tpu as pltpu
from jax.experimental.pallas import tpu_sc as plsc

N = 10000
E = 320000
D = 128
R = 8
RN = R * N
NC = 2
NS = 16
L = 16
NW = NC * NS
EW = E // NW
C = 80
NCH = EW // C
NP = 10240
RPT = NP // NS
CPT = 5120
CNTP = NS * CPT

_MESH = plsc.VectorSubcoreMesh(core_axis_name="c", subcore_axis_name="s")


def _sc_count_body(src_hbm, ty_hbm, dst_hbm, zeros_hbm, counts_hbm,
                   mb, idxb, onesb, cacc):
    c = lax.axis_index("c")
    s = lax.axis_index("s")
    wid = c * NS + s
    base = wid * EW
    pltpu.sync_copy(zeros_hbm.at[pl.ds(s * CPT, CPT)],
                    cacc.at[pl.ds(s * CPT, CPT)])
    for j in range(C // L):
        onesb[pl.ds(j * L, L)] = jnp.full((L,), 1.0, jnp.float32)
    plsc.subcore_barrier()

    def chunk(ci, carry):
        off = base + ci * C
        pltpu.sync_copy(ty_hbm.at[pl.ds(off, C)], mb.at[1])
        pltpu.sync_copy(dst_hbm.at[pl.ds(off, C)], mb.at[2])
        for j in range(C // L):
            ty = mb[1, pl.ds(j * L, L)]
            dv = mb[2, pl.ds(j * L, L)]
            idxb[0, pl.ds(j * L, L)] = ty * N + dv
        pltpu.sync_copy(onesb, cacc.at[idxb.at[0]], add=True)
        return carry

    lax.fori_loop(0, NCH, chunk, 0)
    plsc.subcore_barrier()
    pltpu.sync_copy(cacc.at[pl.ds(s * CPT, CPT)],
                    counts_hbm.at[pl.ds(c * CNTP + s * CPT, CPT)])


_sc_count = pl.kernel(
    _sc_count_body,
    mesh=_MESH,
    out_type=jax.ShapeDtypeStruct((NC * CNTP,), jnp.float32),
    scratch_types=[
        pltpu.VMEM((3, C), jnp.int32),
        pltpu.VMEM((1, C), jnp.int32),
        pltpu.VMEM((C,), jnp.float32),
        pltpu.VMEM_SHARED((CNTP,), jnp.float32),
    ],
)


def _sc_main_body(src_hbm, ty_hbm, dst_hbm, xr_hbm, inv_hbm, zeros_hbm,
                  msg_hbm, mb, idxb, scaleb, rowsb, acc):
    c = lax.axis_index("c")
    s = lax.axis_index("s")
    wid = c * NS + s
    base = wid * EW
    pltpu.sync_copy(zeros_hbm.at[pl.ds(s * RPT, RPT)],
                    acc.at[pl.ds(s * RPT, RPT)])
    plsc.subcore_barrier()

    def chunk(ci, carry):
        off = base + ci * C
        pltpu.sync_copy(src_hbm.at[pl.ds(off, C)], mb.at[0])
        pltpu.sync_copy(ty_hbm.at[pl.ds(off, C)], mb.at[1])
        pltpu.sync_copy(dst_hbm.at[pl.ds(off, C)], mb.at[2])
        for j in range(C // L):
            sv = mb[0, pl.ds(j * L, L)]
            ty = mb[1, pl.ds(j * L, L)]
            dv = mb[2, pl.ds(j * L, L)]
            tn = ty * N
            idxb[0, pl.ds(j * L, L)] = tn + sv
            idxb[1, pl.ds(j * L, L)] = tn + dv
        pltpu.sync_copy(xr_hbm.at[idxb.at[0]], rowsb)
        pltpu.sync_copy(inv_hbm.at[idxb.at[1]], scaleb.at[pl.ds(0, C)])

        def edge(i, ecarry):
            sc = scaleb[pl.ds(i, L)][0]
            for k in range(D // L):
                rowsb[i, pl.ds(k * L, L)] = rowsb[i, pl.ds(k * L, L)] * sc
            return ecarry

        lax.fori_loop(0, C, edge, 0)
        pltpu.sync_copy(rowsb, acc.at[mb.at[2]], add=True)
        return carry

    lax.fori_loop(0, NCH, chunk, 0)
    plsc.subcore_barrier()
    pltpu.sync_copy(acc.at[pl.ds(s * RPT, RPT)],
                    msg_hbm.at[c, pl.ds(s * RPT, RPT)])


_sc_main = pl.kernel(
    _sc_main_body,
    mesh=_MESH,
    out_type=jax.ShapeDtypeStruct((NC, NP, D), jnp.float32),
    scratch_types=[
        pltpu.VMEM((3, C), jnp.int32),
        pltpu.VMEM((2, C), jnp.int32),
        pltpu.VMEM((C + L,), jnp.float32),
        pltpu.VMEM((C, D), jnp.float32),
        pltpu.VMEM_SHARED((NP, D), jnp.float32),
    ],
)


BN = 400


def _mm_body(x_ref, w_ref, o_ref):
    o_ref[0] = jnp.dot(x_ref[...], w_ref[0], preferred_element_type=jnp.float32)


_mm = pl.pallas_call(
    _mm_body,
    grid=(9, N // BN),
    in_specs=[pl.BlockSpec((BN, D), lambda r, i: (i, 0)),
              pl.BlockSpec((1, D, D), lambda r, i: (r, 0, 0))],
    out_specs=pl.BlockSpec((1, BN, D), lambda r, i: (r, i, 0)),
    out_shape=jax.ShapeDtypeStruct((9, N, D), jnp.float32),
)


def _inv_body(c_ref, o_ref):
    csum = c_ref[0] + c_ref[1]
    o_ref[...] = 1.0 / jnp.maximum(csum, 1.0)


_inv = pl.pallas_call(
    _inv_body,
    out_shape=jax.ShapeDtypeStruct((CNTP // D, D), jnp.float32),
)


def _fin_body(b_ref, m_ref, bias_ref, o_ref):
    o_ref[...] = b_ref[...] + m_ref[0] + m_ref[1] + bias_ref[...]


_fin = pl.pallas_call(
    _fin_body,
    grid=(N // BN,),
    in_specs=[pl.BlockSpec((BN, D), lambda i: (i, 0)),
              pl.BlockSpec((2, BN, D), lambda i: (0, i, 0)),
              pl.BlockSpec((1, D), lambda i: (0, 0))],
    out_specs=pl.BlockSpec((BN, D), lambda i: (i, 0)),
    out_shape=jax.ShapeDtypeStruct((N, D), jnp.float32),
)


def kernel(x, edge_index, edge_type, rel_weight, self_loop_weight, bias):
    src = edge_index[0]
    dst = edge_index[1]
    w9 = jnp.concatenate([rel_weight, self_loop_weight[None]], axis=0)
    y9 = _mm(x, w9)
    zeros1 = jnp.zeros((CNTP,), jnp.float32)
    zeros2 = jnp.zeros((NP, D), jnp.float32)
    counts = _sc_count(src, edge_type, dst, zeros1)
    inv = _inv(counts.reshape(NC, CNTP // D, D)).reshape(CNTP)
    msg = _sc_main(src, edge_type, dst, y9.reshape(9 * N, D), inv, zeros2)
    out = _fin(y9[8], msg, bias.reshape(1, D))
    return out


# speedup vs baseline: 43.2986x; 1.1454x over previous
import jax
import jax.numpy as jnp
from jax import lax
from jax.experimental import pallas as pl
from jax.experimental.pallas import tpu as pltpu
from jax.experimental.pallas import tpu_sc as plsc

N = 10000
E = 320000
D = 128
R = 8
RN = R * N
NC = 2
NS = 16
L = 16
NW = NC * NS
C = 128
G = E // C
GBASE = G // NW
GEXTRA = G - GBASE * NW
RSZ = 3 * C
NB = 3
NBM = 2
NP = 10240
RPT = NP // NS
CPT = 5120
CNTP = NS * CPT

_MESH = plsc.VectorSubcoreMesh(core_axis_name="c", subcore_axis_name="s")


def _sc_count_body(meta_hbm, zeros_hbm, counts_hbm,
                   mb, idxb, onesb, cacc, msem, ssem):
    c = lax.axis_index("c")
    s = lax.axis_index("s")
    wid = c * NS + s
    nch = GBASE + jnp.where(wid < GEXTRA, 1, 0)
    pltpu.sync_copy(zeros_hbm.at[pl.ds(s * CPT, CPT)],
                    cacc.at[pl.ds(s * CPT, CPT)])
    for j in range(C // L):
        onesb[pl.ds(j * L, L)] = jnp.full((L,), 1.0, jnp.float32)
    plsc.subcore_barrier()

    def fire_meta(k):
        g = k * NW + wid
        b = k % NB
        pltpu.async_copy(meta_hbm.at[pl.ds(g * RSZ, RSZ)], mb.at[b],
                         msem.at[b])

    def wait_meta(b):
        pltpu.make_async_copy(meta_hbm.at[pl.ds(0, RSZ)], mb.at[b],
                              msem.at[b]).wait()

    fire_meta(0)
    fire_meta(1)

    def step(k, carry):
        b = lax.rem(k, NB)
        wait_meta(b)

        @pl.when(k + 2 < nch)
        def _():
            fire_meta(k + 2)

        @pl.when(k >= NB)
        def _():
            pltpu.make_async_copy(onesb, cacc.at[idxb.at[b]],
                                  ssem.at[b]).wait()

        for j in range(C // L):
            ty = mb[b, pl.ds(C + j * L, L)]
            dv = mb[b, pl.ds(2 * C + j * L, L)]
            idxb[b, pl.ds(j * L, L)] = ty * N + dv
        pltpu.async_copy(onesb, cacc.at[idxb.at[b]], ssem.at[b], add=True)
        return carry

    lax.fori_loop(0, nch, step, 0)
    for b in range(NB):
        pltpu.make_async_copy(onesb, cacc.at[idxb.at[b]], ssem.at[b]).wait()
    plsc.subcore_barrier()
    pltpu.sync_copy(cacc.at[pl.ds(s * CPT, CPT)],
                    counts_hbm.at[pl.ds(c * CNTP + s * CPT, CPT)])


_sc_count = pl.kernel(
    _sc_count_body,
    mesh=_MESH,
    out_type=jax.ShapeDtypeStruct((NC * CNTP,), jnp.float32),
    scratch_types=[
        pltpu.VMEM((NB, RSZ), jnp.int32),
        pltpu.VMEM((NB, C), jnp.int32),
        pltpu.VMEM((C,), jnp.float32),
        pltpu.VMEM_SHARED((CNTP,), jnp.float32),
        pltpu.SemaphoreType.DMA((NB,)),
        pltpu.SemaphoreType.DMA((NB,)),
    ],
)


def _sc_main_body(meta_hbm, xr_hbm, inv_hbm, zeros_hbm, msg_hbm,
                  mb, idxb, scaleb, rowsb, acc, msem, gsem, ssem):
    c = lax.axis_index("c")
    s = lax.axis_index("s")
    wid = c * NS + s
    nch = GBASE + jnp.where(wid < GEXTRA, 1, 0)
    pltpu.sync_copy(zeros_hbm.at[pl.ds(s * RPT, RPT)],
                    acc.at[pl.ds(s * RPT, RPT)])
    plsc.subcore_barrier()

    def fire_meta(k):
        g = k * NW + wid
        b = k % NBM
        pltpu.async_copy(meta_hbm.at[pl.ds(g * RSZ, RSZ)], mb.at[b],
                         msem.at[b])

    def wait_meta(b):
        pltpu.make_async_copy(meta_hbm.at[pl.ds(0, RSZ)], mb.at[b],
                              msem.at[b]).wait()

    def compute_idx(b):
        for j in range(C // L):
            sv = mb[b, pl.ds(j * L, L)]
            ty = mb[b, pl.ds(C + j * L, L)]
            dv = mb[b, pl.ds(2 * C + j * L, L)]
            tn = ty * N
            idxb[3 * b, pl.ds(j * L, L)] = tn + sv
            idxb[3 * b + 1, pl.ds(j * L, L)] = tn + dv
            idxb[3 * b + 2, pl.ds(j * L, L)] = dv

    def fire_gathers(b):
        pltpu.async_copy(xr_hbm.at[idxb.at[3 * b]], rowsb.at[b], gsem.at[b])
        pltpu.async_copy(inv_hbm.at[idxb.at[3 * b + 1]], scaleb.at[b],
                         gsem.at[b])

    def wait_gathers(b):
        pltpu.make_async_copy(xr_hbm.at[idxb.at[3 * b]], rowsb.at[b],
                              gsem.at[b]).wait()
        pltpu.make_async_copy(inv_hbm.at[idxb.at[3 * b + 1]], scaleb.at[b],
                              gsem.at[b]).wait()

    def wait_scatter(b):
        pltpu.make_async_copy(rowsb.at[b], acc.at[idxb.at[3 * b + 2]],
                              ssem.at[b]).wait()

    fire_meta(0)
    fire_meta(1)
    wait_meta(0)
    compute_idx(0)
    fire_gathers(0)

    def step(k, carry):
        b = lax.rem(k, NBM)
        kn = k + 1
        bn = lax.rem(kn, NBM)

        @pl.when(kn < nch)
        def _():
            wait_meta(bn)

            @pl.when(kn >= NBM)
            def _():
                wait_scatter(bn)

            compute_idx(bn)
            fire_gathers(bn)

        @pl.when(k + 2 < nch)
        def _():
            fire_meta(k + 2)

        wait_gathers(b)

        def group(j, gcarry):
            sv16 = scaleb[b, pl.ds(j * L, L)]
            for t in range(L):
                e = j * L + t
                sc = sv16[t]
                for kk in range(D // L):
                    rowsb[b, e, pl.ds(kk * L, L)] = (
                        rowsb[b, e, pl.ds(kk * L, L)] * sc)
            return gcarry

        lax.fori_loop(0, C // L, group, 0)
        pltpu.async_copy(rowsb.at[b], acc.at[idxb.at[3 * b + 2]],
                         ssem.at[b], add=True)
        return carry

    lax.fori_loop(0, nch, step, 0)
    for b in range(NBM):
        wait_scatter(b)
    plsc.subcore_barrier()
    pltpu.sync_copy(acc.at[pl.ds(s * RPT, RPT)],
                    msg_hbm.at[c, pl.ds(s * RPT, RPT)])


_sc_main = pl.kernel(
    _sc_main_body,
    mesh=_MESH,
    out_type=jax.ShapeDtypeStruct((NC, NP, D), jnp.float32),
    scratch_types=[
        pltpu.VMEM((NBM, RSZ), jnp.int32),
        pltpu.VMEM((3 * NBM, C), jnp.int32),
        pltpu.VMEM((NBM + 1, C), jnp.float32),
        pltpu.VMEM((NBM, C, D), jnp.float32),
        pltpu.VMEM_SHARED((NP, D), jnp.float32),
        pltpu.SemaphoreType.DMA((NBM,)),
        pltpu.SemaphoreType.DMA((NBM,)),
        pltpu.SemaphoreType.DMA((NBM,)),
    ],
)


BN = 400


def _mm_body(x_ref, w_ref, o_ref):
    o_ref[0] = jnp.dot(x_ref[...], w_ref[0], preferred_element_type=jnp.float32)


_mm = pl.pallas_call(
    _mm_body,
    grid=(9, N // BN),
    in_specs=[pl.BlockSpec((BN, D), lambda r, i: (i, 0)),
              pl.BlockSpec((1, D, D), lambda r, i: (r, 0, 0))],
    out_specs=pl.BlockSpec((1, BN, D), lambda r, i: (r, i, 0)),
    out_shape=jax.ShapeDtypeStruct((9, N, D), jnp.float32),
)


def _inv_body(c_ref, o_ref):
    csum = c_ref[0] + c_ref[1]
    o_ref[...] = 1.0 / jnp.maximum(csum, 1.0)


_inv = pl.pallas_call(
    _inv_body,
    out_shape=jax.ShapeDtypeStruct((CNTP // D, D), jnp.float32),
)


def _fin_body(b_ref, m_ref, bias_ref, o_ref):
    o_ref[...] = b_ref[...] + m_ref[0] + m_ref[1] + bias_ref[...]


_fin = pl.pallas_call(
    _fin_body,
    grid=(N // BN,),
    in_specs=[pl.BlockSpec((BN, D), lambda i: (i, 0)),
              pl.BlockSpec((2, BN, D), lambda i: (0, i, 0)),
              pl.BlockSpec((1, D), lambda i: (0, 0))],
    out_specs=pl.BlockSpec((BN, D), lambda i: (i, 0)),
    out_shape=jax.ShapeDtypeStruct((N, D), jnp.float32),
)


def kernel(x, edge_index, edge_type, rel_weight, self_loop_weight, bias):
    src = edge_index[0]
    dst = edge_index[1]
    meta = jnp.stack([src.reshape(G, C), edge_type.reshape(G, C),
                      dst.reshape(G, C)], axis=1).reshape(-1)
    w9 = jnp.concatenate([rel_weight, self_loop_weight[None]], axis=0)
    y9 = _mm(x, w9)
    zeros1 = jnp.zeros((CNTP,), jnp.float32)
    zeros2 = jnp.zeros((NP, D), jnp.float32)
    counts = _sc_count(meta, zeros1)
    inv = _inv(counts.reshape(NC, CNTP // D, D)).reshape(CNTP)
    msg = _sc_main(meta, y9.reshape(9 * N, D), inv, zeros2)
    out = _fin(y9[8], msg, bias.reshape(1, D))
    return out


# speedup vs baseline: 58.9021x; 1.3604x over previous
import jax
import jax.numpy as jnp
from jax import lax
from jax.experimental import pallas as pl
from jax.experimental.pallas import tpu as pltpu
from jax.experimental.pallas import tpu_sc as plsc

N = 10000
E = 320000
D = 128
R = 8
RN = R * N
NC = 2
NS = 16
L = 16
NW = NC * NS
C = 128
G = E // C
GBASE = G // NW
GEXTRA = G - GBASE * NW
RSZ = 3 * C
NB = 3
NBM = 2
NP = 10240
RPT = NP // NS
CPT = 5120
CNTP = NS * CPT

_MESH = plsc.VectorSubcoreMesh(core_axis_name="c", subcore_axis_name="s")


def _sc_count_body(meta_hbm, zeros_hbm, counts_hbm,
                   mb, idxb, onesb, cacc, msem, ssem):
    c = lax.axis_index("c")
    s = lax.axis_index("s")
    wid = c * NS + s
    nch = GBASE + jnp.where(wid < GEXTRA, 1, 0)
    pltpu.sync_copy(zeros_hbm.at[pl.ds(s * CPT, CPT)],
                    cacc.at[pl.ds(s * CPT, CPT)])
    for j in range(C // L):
        onesb[pl.ds(j * L, L)] = jnp.full((L,), 1.0, jnp.float32)
    plsc.subcore_barrier()

    def fire_meta(k):
        g = k * NW + wid
        b = k % NB
        pltpu.async_copy(meta_hbm.at[pl.ds(g * RSZ, RSZ)], mb.at[b],
                         msem.at[b])

    def wait_meta(b):
        pltpu.make_async_copy(meta_hbm.at[pl.ds(0, RSZ)], mb.at[b],
                              msem.at[b]).wait()

    fire_meta(0)
    fire_meta(1)

    def step(k, carry):
        b = lax.rem(k, NB)
        wait_meta(b)

        @pl.when(k + 2 < nch)
        def _():
            fire_meta(k + 2)

        @pl.when(k >= NB)
        def _():
            pltpu.make_async_copy(onesb, cacc.at[idxb.at[b]],
                                  ssem.at[b]).wait()

        for j in range(C // L):
            ty = mb[b, pl.ds(C + j * L, L)]
            dv = mb[b, pl.ds(2 * C + j * L, L)]
            idxb[b, pl.ds(j * L, L)] = ty * N + dv
        pltpu.async_copy(onesb, cacc.at[idxb.at[b]], ssem.at[b], add=True)
        return carry

    lax.fori_loop(0, nch, step, 0)
    for b in range(NB):
        pltpu.make_async_copy(onesb, cacc.at[idxb.at[b]], ssem.at[b]).wait()
    plsc.subcore_barrier()
    pltpu.sync_copy(cacc.at[pl.ds(s * CPT, CPT)],
                    counts_hbm.at[pl.ds(c * CNTP + s * CPT, CPT)])


_sc_count = pl.kernel(
    _sc_count_body,
    mesh=_MESH,
    out_type=jax.ShapeDtypeStruct((NC * CNTP,), jnp.float32),
    scratch_types=[
        pltpu.VMEM((NB, RSZ), jnp.int32),
        pltpu.VMEM((NB, C), jnp.int32),
        pltpu.VMEM((C,), jnp.float32),
        pltpu.VMEM_SHARED((CNTP,), jnp.float32),
        pltpu.SemaphoreType.DMA((NB,)),
        pltpu.SemaphoreType.DMA((NB,)),
    ],
)


def _sc_main_body(meta_hbm, xr_hbm, inv_hbm, zeros_hbm, msg_hbm,
                  mb, idxb, scaleb, rowsb, acc, msem, gsem, ssem):
    c = lax.axis_index("c")
    s = lax.axis_index("s")
    wid = c * NS + s
    nch = GBASE + jnp.where(wid < GEXTRA, 1, 0)
    pltpu.sync_copy(zeros_hbm.at[pl.ds(s * RPT, RPT)],
                    acc.at[pl.ds(s * RPT, RPT)])
    plsc.subcore_barrier()

    def fire_meta(k):
        g = k * NW + wid
        b = k % NBM
        pltpu.async_copy(meta_hbm.at[pl.ds(g * RSZ, RSZ)], mb.at[b],
                         msem.at[b])

    def wait_meta(b):
        pltpu.make_async_copy(meta_hbm.at[pl.ds(0, RSZ)], mb.at[b],
                              msem.at[b]).wait()

    def compute_idx(b):
        for j in range(C // L):
            sv = mb[b, pl.ds(j * L, L)]
            ty = mb[b, pl.ds(C + j * L, L)]
            dv = mb[b, pl.ds(2 * C + j * L, L)]
            tn = ty * N
            idxb[3 * b, pl.ds(j * L, L)] = tn + sv
            idxb[3 * b + 1, pl.ds(j * L, L)] = tn + dv
            idxb[3 * b + 2, pl.ds(j * L, L)] = dv

    def fire_gathers(b):
        pltpu.async_copy(xr_hbm.at[idxb.at[3 * b]], rowsb.at[b], gsem.at[b])
        pltpu.async_copy(inv_hbm.at[idxb.at[3 * b + 1]], scaleb.at[b],
                         gsem.at[b])

    def wait_gathers(b):
        pltpu.make_async_copy(xr_hbm.at[idxb.at[3 * b]], rowsb.at[b],
                              gsem.at[b]).wait()
        pltpu.make_async_copy(inv_hbm.at[idxb.at[3 * b + 1]], scaleb.at[b],
                              gsem.at[b]).wait()

    def wait_scatter(b):
        pltpu.make_async_copy(rowsb.at[b], acc.at[idxb.at[3 * b + 2]],
                              ssem.at[b]).wait()

    fire_meta(0)
    fire_meta(1)
    wait_meta(0)
    compute_idx(0)
    fire_gathers(0)

    def step(k, carry):
        b = lax.rem(k, NBM)
        kn = k + 1
        bn = lax.rem(kn, NBM)

        @pl.when(kn < nch)
        def _():
            wait_meta(bn)

            @pl.when(kn >= NBM)
            def _():
                wait_scatter(bn)

            compute_idx(bn)
            fire_gathers(bn)

        @pl.when(k + 2 < nch)
        def _():
            fire_meta(k + 2)

        wait_gathers(b)

        @plsc.parallel_loop(0, C // L, 1, unroll=2)
        def _scale(j):
            sv16 = scaleb[b, pl.ds(j * L, L)]
            for t in range(L):
                e = j * L + t
                sc = sv16[t]
                for kk in range(D // L):
                    rowsb[b, e, pl.ds(kk * L, L)] = (
                        rowsb[b, e, pl.ds(kk * L, L)] * sc)
        pltpu.async_copy(rowsb.at[b], acc.at[idxb.at[3 * b + 2]],
                         ssem.at[b], add=True)
        return carry

    lax.fori_loop(0, nch, step, 0)
    for b in range(NBM):
        wait_scatter(b)
    plsc.subcore_barrier()
    pltpu.sync_copy(acc.at[pl.ds(s * RPT, RPT)],
                    msg_hbm.at[c, pl.ds(s * RPT, RPT)])


_sc_main = pl.kernel(
    _sc_main_body,
    mesh=_MESH,
    out_type=jax.ShapeDtypeStruct((NC, NP, D), jnp.float32),
    scratch_types=[
        pltpu.VMEM((NBM, RSZ), jnp.int32),
        pltpu.VMEM((3 * NBM, C), jnp.int32),
        pltpu.VMEM((NBM + 1, C), jnp.float32),
        pltpu.VMEM((NBM, C, D), jnp.float32),
        pltpu.VMEM_SHARED((NP, D), jnp.float32),
        pltpu.SemaphoreType.DMA((NBM,)),
        pltpu.SemaphoreType.DMA((NBM,)),
        pltpu.SemaphoreType.DMA((NBM,)),
    ],
)


BN = 400


def _mm_body(x_ref, w_ref, o_ref):
    o_ref[0] = jnp.dot(x_ref[...], w_ref[0], preferred_element_type=jnp.float32)


_mm = pl.pallas_call(
    _mm_body,
    grid=(9, N // BN),
    in_specs=[pl.BlockSpec((BN, D), lambda r, i: (i, 0)),
              pl.BlockSpec((1, D, D), lambda r, i: (r, 0, 0))],
    out_specs=pl.BlockSpec((1, BN, D), lambda r, i: (r, i, 0)),
    out_shape=jax.ShapeDtypeStruct((9, N, D), jnp.float32),
)


def _inv_body(c_ref, o_ref):
    csum = c_ref[0] + c_ref[1]
    o_ref[...] = 1.0 / jnp.maximum(csum, 1.0)


_inv = pl.pallas_call(
    _inv_body,
    out_shape=jax.ShapeDtypeStruct((CNTP // D, D), jnp.float32),
)


def _fin_body(b_ref, m_ref, bias_ref, o_ref):
    o_ref[...] = b_ref[...] + m_ref[0] + m_ref[1] + bias_ref[...]


_fin = pl.pallas_call(
    _fin_body,
    grid=(N // BN,),
    in_specs=[pl.BlockSpec((BN, D), lambda i: (i, 0)),
              pl.BlockSpec((2, BN, D), lambda i: (0, i, 0)),
              pl.BlockSpec((1, D), lambda i: (0, 0))],
    out_specs=pl.BlockSpec((BN, D), lambda i: (i, 0)),
    out_shape=jax.ShapeDtypeStruct((N, D), jnp.float32),
)


def kernel(x, edge_index, edge_type, rel_weight, self_loop_weight, bias):
    src = edge_index[0]
    dst = edge_index[1]
    meta = jnp.stack([src.reshape(G, C), edge_type.reshape(G, C),
                      dst.reshape(G, C)], axis=1).reshape(-1)
    w9 = jnp.concatenate([rel_weight, self_loop_weight[None]], axis=0)
    y9 = _mm(x, w9)
    zeros1 = jnp.zeros((CNTP,), jnp.float32)
    zeros2 = jnp.zeros((NP, D), jnp.float32)
    counts = _sc_count(meta, zeros1)
    inv = _inv(counts.reshape(NC, CNTP // D, D)).reshape(CNTP)
    msg = _sc_main(meta, y9.reshape(9 * N, D), inv, zeros2)
    out = _fin(y9[8], msg, bias.reshape(1, D))
    return out
